# Initial kernel scaffold; baseline (speedup 1.0000x reference)
import jax
import jax.numpy as jnp
from jax.experimental import pallas as pl


def kernel(action_idx, table):
    raise NotImplementedError("write your pallas kernel here")


# speedup vs baseline: 1.0944x; 1.0944x over previous
import functools

import jax
import jax.numpy as jnp
from jax import lax
from jax.experimental import pallas as pl
from jax.experimental.pallas import tpu as pltpu
from jax.experimental.pallas import tpu_sc as plsc

_NUM_CORES = 2
_NUM_SUBCORES = 16
_NUM_WORKERS = _NUM_CORES * _NUM_SUBCORES
_LANES = 16

_CHUNK = 1024
_GATHER = 128
_NG = _CHUNK // _GATHER


@functools.lru_cache(maxsize=None)
def _make_emb(B: int, D: int):
    assert B % (_NUM_WORKERS * _CHUNK) == 0
    b_per_w = B // _NUM_WORKERS
    n_chunks = b_per_w // _CHUNK
    mesh = plsc.VectorSubcoreMesh(core_axis_name="c", subcore_axis_name="s")

    @functools.partial(
        pl.kernel,
        out_type=jax.ShapeDtypeStruct((B, D), jnp.float32),
        mesh=mesh,
        scratch_types=[
            pltpu.VMEM((_CHUNK,), jnp.int32),
            pltpu.VMEM((_CHUNK, D), jnp.float32),
            pltpu.SemaphoreType.DMA,
        ],
        compiler_params=pltpu.CompilerParams(use_tc_tiling_on_sc=False),
    )
    def emb(idx_hbm, table_hbm, out_hbm, idx_v, rows_v, sem):
        wid = lax.axis_index("s") * _NUM_CORES + lax.axis_index("c")
        base_w = wid * b_per_w

        def chunk_body(s, carry):
            base = base_w + s * _CHUNK
            pltpu.sync_copy(idx_hbm.at[pl.ds(base, _CHUNK)], idx_v)
            for t in range(_CHUNK // _LANES):
                v = idx_v[pl.ds(t * _LANES, _LANES)]
                idx_v[pl.ds(t * _LANES, _LANES)] = (
                    jnp.where(v == -100, 0, v) + 1
                )
            copies = [
                pltpu.async_copy(
                    table_hbm.at[idx_v.at[pl.ds(g * _GATHER, _GATHER)]],
                    rows_v.at[pl.ds(g * _GATHER, _GATHER)],
                    sem,
                )
                for g in range(_NG)
            ]
            for c in copies:
                c.wait()
            pltpu.sync_copy(rows_v, out_hbm.at[pl.ds(base, _CHUNK)])
            return carry

        lax.fori_loop(0, n_chunks, chunk_body, 0)

    return emb


def kernel(action_idx, table):
    B = action_idx.size
    D = table.shape[1]
    out = _make_emb(B, D)(action_idx.reshape(B), table)
    return out.reshape(*action_idx.shape, D)


# speedup vs baseline: 1.1140x; 1.0179x over previous
import functools

import jax
import jax.numpy as jnp
from jax import lax
from jax.experimental import pallas as pl
from jax.experimental.pallas import tpu as pltpu
from jax.experimental.pallas import tpu_sc as plsc

_NUM_CORES = 2
_NUM_SUBCORES = 16
_NUM_WORKERS = _NUM_CORES * _NUM_SUBCORES
_LANES = 16

_CHUNK = 1280
_GATHER = 128
_NG = _CHUNK // _GATHER


@functools.lru_cache(maxsize=None)
def _make_emb(B: int, D: int):
    assert B % (_NUM_WORKERS * 2 * _CHUNK) == 0
    b_per_w = B // _NUM_WORKERS
    n_chunks = b_per_w // _CHUNK
    n_iters = n_chunks // 2
    chunk_bytes = _CHUNK * D * 4
    mesh = plsc.VectorSubcoreMesh(core_axis_name="c", subcore_axis_name="s")

    @functools.partial(
        pl.kernel,
        out_type=jax.ShapeDtypeStruct((B, D), jnp.float32),
        mesh=mesh,
        scratch_types=[
            pltpu.VMEM((b_per_w,), jnp.int32),
            pltpu.VMEM((_CHUNK, D), jnp.float32),
            pltpu.VMEM((_CHUNK, D), jnp.float32),
            pltpu.SemaphoreType.DMA,
            pltpu.SemaphoreType.DMA,
            pltpu.SemaphoreType.DMA,
            pltpu.SemaphoreType.DMA,
        ],
        compiler_params=pltpu.CompilerParams(use_tc_tiling_on_sc=False),
    )
    def emb(idx_hbm, table_hbm, out_hbm, idx_all, rows_a, rows_b,
            semg_a, semg_b, semst_a, semst_b):
        wid = lax.axis_index("s") * _NUM_CORES + lax.axis_index("c")
        base_w = wid * b_per_w

        def transform(s):
            def t_body(i, carry):
                off = s * _CHUNK + i * _LANES
                v = idx_all[pl.ds(off, _LANES)]
                idx_all[pl.ds(off, _LANES)] = jnp.where(v == -100, 0, v) + 1
                return carry
            lax.fori_loop(0, _CHUNK // _LANES, t_body, 0)

        def fire_gathers(s, rows, sem):
            for g in range(_NG):
                pltpu.async_copy(
                    table_hbm.at[idx_all.at[pl.ds(s * _CHUNK + g * _GATHER,
                                                  _GATHER)]],
                    rows.at[pl.ds(g * _GATHER, _GATHER)],
                    sem,
                )

        def drain(sem, rows):
            pltpu.make_async_copy(
                out_hbm.at[pl.ds(base_w, _CHUNK)], rows, sem).wait()

        def fire_store(s, rows, sem):
            pltpu.async_copy(
                rows, out_hbm.at[pl.ds(base_w + s * _CHUNK, _CHUNK)], sem)

        def drain_store(sem):
            pltpu.make_async_copy(
                rows_a, out_hbm.at[pl.ds(base_w, _CHUNK)], sem).wait()

        pltpu.sync_copy(idx_hbm.at[pl.ds(base_w, b_per_w)], idx_all)
        transform(0)
        fire_gathers(0, rows_a, semg_a)

        def body(t, carry):
            s0 = 2 * t
            s1 = s0 + 1
            transform(s1)

            @pl.when(t > 0)
            def _():
                drain_store(semst_b)
            fire_gathers(s1, rows_b, semg_b)
            drain(semg_a, rows_a)
            fire_store(s0, rows_a, semst_a)

            @pl.when(t < n_iters - 1)
            def _():
                transform(s0 + 2)
                drain_store(semst_a)
                fire_gathers(s0 + 2, rows_a, semg_a)
            drain(semg_b, rows_b)
            fire_store(s1, rows_b, semst_b)
            return carry

        lax.fori_loop(0, n_iters, body, 0)
        drain_store(semst_a)
        drain_store(semst_b)

    return emb


def kernel(action_idx, table):
    B = action_idx.size
    D = table.shape[1]
    out = _make_emb(B, D)(action_idx.reshape(B), table)
    return out.reshape(*action_idx.shape, D)


# speedup vs baseline: 1.8046x; 1.6199x over previous
import functools

import jax
import jax.numpy as jnp
from jax import lax
from jax.experimental import pallas as pl
from jax.experimental.pallas import tpu as pltpu
from jax.experimental.pallas import tpu_sc as plsc

_NUM_CORES = 2
_NUM_SUBCORES = 16
_NUM_WORKERS = _NUM_CORES * _NUM_SUBCORES
_LANES = 16

_RCHUNK = 16
_HPAD = 56


@functools.lru_cache(maxsize=None)
def _make_emb(B: int, H: int, D: int):
    assert B % (_NUM_WORKERS * 2 * _RCHUNK) == 0
    r_per_w = B // _NUM_WORKERS
    n_chunks = r_per_w // _RCHUNK
    n_iters = n_chunks // 2
    assert _LANES <= H <= _HPAD
    mesh = plsc.VectorSubcoreMesh(core_axis_name="c", subcore_axis_name="s")

    @functools.partial(
        pl.kernel,
        out_type=jax.ShapeDtypeStruct((B, H, D), jnp.float32),
        mesh=mesh,
        scratch_types=[
            pltpu.VMEM((r_per_w, H), jnp.int32),
            pltpu.VMEM((r_per_w, H), jnp.int32),
            pltpu.VMEM((_RCHUNK, H, D), jnp.float32),
            pltpu.VMEM((_RCHUNK, H, D), jnp.float32),
            pltpu.SemaphoreType.DMA,
            pltpu.SemaphoreType.DMA,
            pltpu.SemaphoreType.DMA,
            pltpu.SemaphoreType.DMA,
        ],
        compiler_params=pltpu.CompilerParams(use_tc_tiling_on_sc=False),
    )
    def emb(idx_hbm, table_hbm, out_hbm, idx_v, safe_v, rows_a, rows_b,
            semg_a, semg_b, semst_a, semst_b):
        wid = lax.axis_index("s") * _NUM_CORES + lax.axis_index("c")
        base_w = wid * r_per_w

        pltpu.sync_copy(idx_hbm.at[pl.ds(base_w, r_per_w)], idx_v)

        offsets = sorted(set(range(0, H - _LANES + 1, _LANES)) | {H - _LANES})

        def t_body(r, carry):
            for o in offsets:
                v = idx_v[r, pl.ds(o, _LANES)]
                safe_v[r, pl.ds(o, _LANES)] = jnp.where(v == -100, 0, v) + 1
            return carry

        lax.fori_loop(0, r_per_w, t_body, 0)

        def fire_gathers(s, rows, sem):
            for k in range(_RCHUNK):
                pltpu.async_copy(
                    table_hbm.at[safe_v.at[s * _RCHUNK + k]],
                    rows.at[k],
                    sem,
                )

        def drain(sem, rows):
            pltpu.make_async_copy(
                out_hbm.at[pl.ds(base_w, _RCHUNK)], rows, sem).wait()

        def fire_store(s, rows, sem):
            pltpu.async_copy(
                rows, out_hbm.at[pl.ds(base_w + s * _RCHUNK, _RCHUNK)], sem)

        def drain_store(sem):
            pltpu.make_async_copy(
                rows_a, out_hbm.at[pl.ds(base_w, _RCHUNK)], sem).wait()

        fire_gathers(0, rows_a, semg_a)

        def body(t, carry):
            s0 = 2 * t
            s1 = s0 + 1

            @pl.when(t > 0)
            def _():
                drain_store(semst_b)
            fire_gathers(s1, rows_b, semg_b)
            drain(semg_a, rows_a)
            fire_store(s0, rows_a, semst_a)

            @pl.when(t < n_iters - 1)
            def _():
                drain_store(semst_a)
                fire_gathers(s0 + 2, rows_a, semg_a)
            drain(semg_b, rows_b)
            fire_store(s1, rows_b, semst_b)
            return carry

        lax.fori_loop(0, n_iters, body, 0)
        drain_store(semst_a)
        drain_store(semst_b)

    return emb


def kernel(action_idx, table):
    B, H = action_idx.shape
    D = table.shape[1]
    return _make_emb(B, H, D)(action_idx, table)
